# baseline (device time: 7846 ns/iter reference)
import jax
import jax.numpy as jnp
from jax import lax
from jax.experimental import pallas as pl
from jax.experimental.pallas import tpu as pltpu

NBLK = 4


def kernel(x, dy, gamma):
    m, d = x.shape
    bm = m // NBLK

    def body(
        x_hbm,
        dy_hbm,
        out_ref,
        xb,
        dyb,
        comm_ref,
        x_sems,
        y_sems,
        send_sem,
        recv_sem,
    ):
        my_x = lax.axis_index("x")
        my_y = lax.axis_index("y")
        my_z = lax.axis_index("z")
        partner = (my_x, my_y, 1 - my_z)

        barrier_sem = pltpu.get_barrier_semaphore()
        pl.semaphore_signal(
            barrier_sem,
            inc=1,
            device_id=partner,
            device_id_type=pl.DeviceIdType.MESH,
        )

        def start_copy(k):
            slot = k % 2
            cx = pltpu.make_async_copy(
                x_hbm.at[pl.ds(k * bm, bm), :], xb.at[slot], x_sems.at[slot]
            )
            cy = pltpu.make_async_copy(
                dy_hbm.at[pl.ds(k * bm, bm), :], dyb.at[slot], y_sems.at[slot]
            )
            cx.start()
            cy.start()
            return cx, cy

        copies = [None] * NBLK
        copies[0] = start_copy(0)

        dgamma = jnp.zeros((d,), jnp.float32)
        dbeta = jnp.zeros((d,), jnp.float32)
        for k in range(NBLK):
            if k + 1 < NBLK:
                copies[k + 1] = start_copy(k + 1)
            cx, cy = copies[k]
            cx.wait()
            cy.wait()
            xv = xb[k % 2]
            dyv = dyb[k % 2]
            mu = jnp.mean(xv, axis=1, keepdims=True)
            ex2 = jnp.mean(xv * xv, axis=1, keepdims=True)
            rstd = lax.rsqrt(ex2 - mu * mu + 1e-5)
            dys = dyv * rstd
            dgamma = dgamma + jnp.sum(dys * xv, axis=0) - jnp.sum(dys * mu, axis=0)
            dbeta = dbeta + jnp.sum(dyv, axis=0)

        comm_ref[0, :, :] = jnp.stack([dgamma, dbeta])

        pl.semaphore_wait(barrier_sem, 1)

        rdma = pltpu.make_async_remote_copy(
            src_ref=comm_ref.at[0],
            dst_ref=comm_ref.at[1],
            send_sem=send_sem,
            recv_sem=recv_sem,
            device_id=partner,
            device_id_type=pl.DeviceIdType.MESH,
        )
        rdma.start()
        rdma.wait()

        out_ref[:, :] = comm_ref[0] + comm_ref[1]

    return pl.pallas_call(
        body,
        out_shape=jax.ShapeDtypeStruct((2, d), jnp.float32),
        in_specs=[
            pl.BlockSpec(memory_space=pltpu.MemorySpace.HBM),
            pl.BlockSpec(memory_space=pltpu.MemorySpace.HBM),
        ],
        out_specs=pl.BlockSpec(memory_space=pltpu.VMEM),
        scratch_shapes=[
            pltpu.VMEM((2, bm, d), jnp.float32),
            pltpu.VMEM((2, bm, d), jnp.float32),
            pltpu.VMEM((2, 2, d), jnp.float32),
            pltpu.SemaphoreType.DMA((2,)),
            pltpu.SemaphoreType.DMA((2,)),
            pltpu.SemaphoreType.DMA,
            pltpu.SemaphoreType.DMA,
        ],
        compiler_params=pltpu.CompilerParams(collective_id=0),
    )(x, dy)


# device time: 7762 ns/iter; 1.0108x vs baseline; 1.0108x over previous
import jax
import jax.numpy as jnp
from jax import lax
from jax.experimental import pallas as pl
from jax.experimental.pallas import tpu as pltpu

NBLK = 4


def kernel(x, dy, gamma):
    m, d = x.shape
    bm = m // NBLK

    def body(x_ref, dy_ref, out_ref, acc_ref, comm_ref, send_sem, recv_sem):
        k = pl.program_id(0)
        my_x = lax.axis_index("x")
        my_y = lax.axis_index("y")
        my_z = lax.axis_index("z")
        partner = (my_x, my_y, 1 - my_z)
        barrier_sem = pltpu.get_barrier_semaphore()

        @pl.when(k == 0)
        def _():
            pl.semaphore_signal(
                barrier_sem,
                inc=1,
                device_id=partner,
                device_id_type=pl.DeviceIdType.MESH,
            )

        xv = x_ref[:, :]
        dyv = dy_ref[:, :]
        mu = jnp.mean(xv, axis=1, keepdims=True)
        ex2 = jnp.mean(xv * xv, axis=1, keepdims=True)
        rstd = lax.rsqrt(ex2 - mu * mu + 1e-5)
        dys = dyv * rstd
        dgamma = jnp.sum(dys * xv, axis=0) - jnp.sum(dys * mu, axis=0)
        dbeta = jnp.sum(dyv, axis=0)
        blk = jnp.stack([dgamma, dbeta])

        @pl.when(k == 0)
        def _():
            acc_ref[:, :] = blk

        @pl.when(k > 0)
        def _():
            acc_ref[:, :] = acc_ref[:, :] + blk

        @pl.when(k == NBLK - 1)
        def _():
            comm_ref[0, :, :] = acc_ref[:, :]
            pl.semaphore_wait(barrier_sem, 1)
            rdma = pltpu.make_async_remote_copy(
                src_ref=comm_ref.at[0],
                dst_ref=comm_ref.at[1],
                send_sem=send_sem,
                recv_sem=recv_sem,
                device_id=partner,
                device_id_type=pl.DeviceIdType.MESH,
            )
            rdma.start()
            rdma.wait()
            out_ref[:, :] = comm_ref[0] + comm_ref[1]

    return pl.pallas_call(
        body,
        grid=(NBLK,),
        out_shape=jax.ShapeDtypeStruct((2, d), jnp.float32),
        in_specs=[
            pl.BlockSpec((bm, d), lambda k: (k, 0)),
            pl.BlockSpec((bm, d), lambda k: (k, 0)),
        ],
        out_specs=pl.BlockSpec((2, d), lambda k: (0, 0)),
        scratch_shapes=[
            pltpu.VMEM((2, d), jnp.float32),
            pltpu.VMEM((2, 2, d), jnp.float32),
            pltpu.SemaphoreType.DMA,
            pltpu.SemaphoreType.DMA,
        ],
        compiler_params=pltpu.CompilerParams(collective_id=0),
    )(x, dy)


# device time: 7390 ns/iter; 1.0617x vs baseline; 1.0503x over previous
import jax
import jax.numpy as jnp
from jax import lax
from jax.experimental import pallas as pl
from jax.experimental.pallas import tpu as pltpu


def kernel(x, dy, gamma):
    m, d = x.shape

    def body(x_ref, dy_ref, out_ref, comm_ref, send_sem, recv_sem):
        my_x = lax.axis_index("x")
        my_y = lax.axis_index("y")
        my_z = lax.axis_index("z")
        partner = (my_x, my_y, 1 - my_z)

        barrier_sem = pltpu.get_barrier_semaphore()
        pl.semaphore_signal(
            barrier_sem,
            inc=1,
            device_id=partner,
            device_id_type=pl.DeviceIdType.MESH,
        )

        xv = x_ref[:, :]
        dyv = dy_ref[:, :]
        mu = jnp.mean(xv, axis=1, keepdims=True)
        ex2 = jnp.mean(xv * xv, axis=1, keepdims=True)
        var = ex2 - mu * mu
        rstd = lax.rsqrt(var + 1e-5)
        dys = dyv * rstd
        dgamma = jnp.sum(dys * xv, axis=0) - jnp.sum(dys * mu, axis=0)
        dbeta = jnp.sum(dyv, axis=0)
        comm_ref[0, :, :] = jnp.stack([dgamma, dbeta])

        pl.semaphore_wait(barrier_sem, 1)

        rdma = pltpu.make_async_remote_copy(
            src_ref=comm_ref.at[0],
            dst_ref=comm_ref.at[1],
            send_sem=send_sem,
            recv_sem=recv_sem,
            device_id=partner,
            device_id_type=pl.DeviceIdType.MESH,
        )
        rdma.start()
        rdma.wait()

        out_ref[:, :] = comm_ref[0] + comm_ref[1]

    return pl.pallas_call(
        body,
        out_shape=jax.ShapeDtypeStruct((2, d), jnp.float32),
        in_specs=[
            pl.BlockSpec(memory_space=pltpu.VMEM),
            pl.BlockSpec(memory_space=pltpu.VMEM),
        ],
        out_specs=pl.BlockSpec(memory_space=pltpu.VMEM),
        scratch_shapes=[
            pltpu.VMEM((2, 2, d), jnp.float32),
            pltpu.SemaphoreType.DMA,
            pltpu.SemaphoreType.DMA,
        ],
        compiler_params=pltpu.CompilerParams(collective_id=0),
    )(x, dy)
